# Initial kernel scaffold; baseline (speedup 1.0000x reference)
import jax
import jax.numpy as jnp
from jax.experimental import pallas as pl


def kernel(x):
    raise NotImplementedError("write your pallas kernel here")


# speedup vs baseline: 4.8463x; 4.8463x over previous
import functools

import jax
import jax.numpy as jnp
from jax.experimental import pallas as pl

_TOP_K = 64


def _topk_mask_kernel(x_ref, o_ref, *, k):
    x = x_ref[...]
    u = jax.lax.bitcast_convert_type(x, jnp.uint32)
    top = jnp.uint32(0x80000000)
    key = jnp.where(u >= top, ~u, u | top)

    rows = x.shape[0]
    lo = jnp.zeros((rows, 1), jnp.uint32)
    for b in range(31, -1, -1):
        cand = lo | jnp.uint32(1 << b)
        cnt = jnp.sum((key >= cand).astype(jnp.int32), axis=1, keepdims=True)
        lo = jnp.where(cnt >= k, cand, lo)

    ut = jnp.where(lo >= top, lo ^ top, ~lo)
    t = jax.lax.bitcast_convert_type(ut, jnp.float32)

    gt = x > t
    eq = x == t
    n_gt = jnp.sum(gt.astype(jnp.int32), axis=1, keepdims=True)
    n_eq = jnp.sum(eq.astype(jnp.int32), axis=1, keepdims=True)
    exact = jnp.sum(((n_gt + n_eq) > k).astype(jnp.int32)) == 0

    @pl.when(exact)
    def _():
        o_ref[...] = jnp.where(x >= t, x, 0.0)

    @pl.when(jnp.logical_not(exact))
    def _():
        n_keep = k - n_gt
        idx = jax.lax.broadcasted_iota(jnp.int32, x.shape, 1)
        cut = jnp.zeros((rows, 1), jnp.int32)
        for b in range(16, -1, -1):
            cand = cut | jnp.int32(1 << b)
            cnt_lt = jnp.sum(
                (eq & (idx < cand)).astype(jnp.int32), axis=1, keepdims=True
            )
            cut = jnp.where(cnt_lt <= n_keep, cand, cut)
        keep = gt | (eq & (idx < cut))
        o_ref[...] = jnp.where(keep, x, 0.0)


def kernel(x):
    bsz, d_sae = x.shape
    k = min(_TOP_K, d_sae)
    rows_per_block = 8
    grid = bsz // rows_per_block
    return pl.pallas_call(
        functools.partial(_topk_mask_kernel, k=k),
        out_shape=jax.ShapeDtypeStruct((bsz, d_sae), x.dtype),
        grid=(grid,),
        in_specs=[pl.BlockSpec((rows_per_block, d_sae), lambda i: (i, 0))],
        out_specs=pl.BlockSpec((rows_per_block, d_sae), lambda i: (i, 0)),
    )(x)
